# Initial kernel scaffold; baseline (speedup 1.0000x reference)
import jax
import jax.numpy as jnp
from jax.experimental import pallas as pl


def kernel(x, edge_index, edge_attr, W1, b1, W2, b2, U1, bu1, U2, bu2):
    raise NotImplementedError("write your pallas kernel here")


# speedup vs baseline: 2.2985x; 2.2985x over previous
import functools

import jax
import jax.numpy as jnp
from jax import lax
from jax.experimental import pallas as pl
from jax.experimental.pallas import tpu as pltpu
from jax.experimental.pallas import tpu_sc as plsc

N_NODES = 10000
N_EDGES = 320000
D = 128

NC = 2
NS = 16
L = 16
NW = NC * NS
NP = 10240
CB = NP // D
EP = 327680
EPW = EP // NW
K = 64
NCHUNK = EPW // K
CZ = NP // NS
ZCH = 64

_PREC = lax.Precision.HIGHEST


def _nodeproj_body(x_ref, wa_ref, wb_ref, xa_ref, xb_ref):
    xv = x_ref[...]
    xa_ref[...] = jnp.dot(xv, wa_ref[...], preferred_element_type=jnp.float32,
                          precision=_PREC)
    xb_ref[...] = jnp.dot(xv, wb_ref[...], preferred_element_type=jnp.float32,
                          precision=_PREC)


def _node_proj(x_pad, w1a, w1b):
    tn = 2048
    return pl.pallas_call(
        _nodeproj_body,
        grid=(NP // tn,),
        in_specs=[
            pl.BlockSpec((tn, D), lambda i: (i, 0)),
            pl.BlockSpec((D, D), lambda i: (0, 0)),
            pl.BlockSpec((D, D), lambda i: (0, 0)),
        ],
        out_specs=[
            pl.BlockSpec((tn, D), lambda i: (i, 0)),
            pl.BlockSpec((tn, D), lambda i: (i, 0)),
        ],
        out_shape=[
            jax.ShapeDtypeStruct((NP, D), jnp.float32),
            jax.ShapeDtypeStruct((NP, D), jnp.float32),
        ],
    )(x_pad, w1a, w1b)


def _edgeproj_body(e_ref, w_ref, b_ref, g_ref):
    g_ref[...] = jnp.dot(e_ref[...], w_ref[...], preferred_element_type=jnp.float32,
                         precision=_PREC) + b_ref[...]


def _edge_proj(edge_attr, w1c, b1):
    te = 2560
    n_in = N_EDGES // te
    return pl.pallas_call(
        _edgeproj_body,
        grid=(EP // te,),
        in_specs=[
            pl.BlockSpec((te, D), lambda i: (jnp.minimum(i, n_in - 1), 0)),
            pl.BlockSpec((D, D), lambda i: (0, 0)),
            pl.BlockSpec((1, D), lambda i: (0, 0)),
        ],
        out_specs=pl.BlockSpec((te, D), lambda i: (i, 0)),
        out_shape=jax.ShapeDtypeStruct((EP, D), jnp.float32),
    )(edge_attr, w1c, b1.reshape(1, D))


def _sc_body(row_hbm, col_hbm, xa_hbm, xb_hbm, g_hbm,
             out_hbm, cnt_hbm,
             rowi, coli, cidx, buf_a, buf_b, buf_g, obuf, cntv, acc,
             sem_a, sem_b, sem_g):
    c = lax.axis_index("c")
    s = lax.axis_index("s")
    base = (c * NS + s) * EPW

    zv = jnp.zeros((L,), jnp.float32)

    def _zrow(i, _):
        for j in range(D // L):
            obuf[i, pl.ds(j * L, L)] = zv
        return 0
    lax.fori_loop(0, K, _zrow, 0)

    def _zacc(t, _):
        pltpu.sync_copy(obuf, acc.at[pl.ds(s * CZ + t * ZCH, ZCH)])
        return 0
    lax.fori_loop(0, CZ // ZCH, _zacc, 0)

    def _zcnt(t, _):
        for j in range(D // L):
            cntv[t, pl.ds(j * L, L)] = zv
        return 0
    lax.fori_loop(0, CB, _zcnt, 0)

    plsc.subcore_barrier()

    ones = jnp.ones((L,), jnp.float32)

    def _chunk(k, _):
        eb = base + k * K
        pltpu.sync_copy(row_hbm.at[pl.ds(eb, K)], rowi)
        pltpu.sync_copy(col_hbm.at[pl.ds(eb, K)], coli)
        ca = pltpu.async_copy(xa_hbm.at[rowi], buf_a, sem_a)
        cb = pltpu.async_copy(xb_hbm.at[coli], buf_b, sem_b)
        cg = pltpu.async_copy(g_hbm.at[pl.ds(eb, K)], buf_g, sem_g)
        ca.wait()
        cb.wait()
        cg.wait()

        def _crow(i, _):
            for j in range(D // L):
                sl = pl.ds(j * L, L)
                t = buf_a[i, sl] + buf_b[i, sl] + buf_g[i, sl]
                obuf[i, sl] = jnp.maximum(t, 0.0)
            return 0
        lax.fori_loop(0, K, _crow, 0)

        def _group(g, _):
            cv = coli[pl.ds(g * L, L)]
            plsc.addupdate_scatter(
                cntv, [lax.shift_right_logical(cv, 7),
                       lax.bitwise_and(cv, jnp.int32(127))], ones)
            return 0
        lax.fori_loop(0, K // L, _group, 0)

        pltpu.sync_copy(obuf, acc.at[coli], add=True)
        return 0
    lax.fori_loop(0, NCHUNK, _chunk, 0)

    plsc.subcore_barrier()

    def _dump(t, _):
        r0 = s * CZ + t * ZCH
        pltpu.sync_copy(acc.at[pl.ds(r0, ZCH)], obuf)
        pltpu.sync_copy(obuf, out_hbm.at[c, pl.ds(r0, ZCH)])
        return 0
    lax.fori_loop(0, CZ // ZCH, _dump, 0)

    plsc.subcore_barrier()

    @pl.when(s == 0)
    def _zero_cnt_rows():
        def _z2(i, _):
            for j in range(D // L):
                obuf[i, pl.ds(j * L, L)] = zv
            return 0
        lax.fori_loop(0, K, _z2, 0)

        def _zc(t, _):
            pltpu.sync_copy(obuf.at[pl.ds(0, CB // 2)],
                            acc.at[pl.ds(t * (CB // 2), CB // 2)])
            return 0
        lax.fori_loop(0, 2, _zc, 0)

    plsc.subcore_barrier()

    def _credu(h, _):
        cidx[...] = jnp.arange(L, dtype=jnp.int32) + h * L
        pltpu.sync_copy(cntv.at[pl.ds(h * L, L)], acc.at[cidx], add=True)
        return 0
    lax.fori_loop(0, CB // L, _credu, 0)
    plsc.subcore_barrier()

    @pl.when(s == 0)
    def _dump_cnt():
        def _dc(t, _):
            r0 = t * (CB // 2)
            pltpu.sync_copy(acc.at[pl.ds(r0, CB // 2)],
                            obuf.at[pl.ds(0, CB // 2)])
            pltpu.sync_copy(obuf.at[pl.ds(0, CB // 2)],
                            cnt_hbm.at[c, pl.ds(r0, CB // 2)])
            return 0
        lax.fori_loop(0, 2, _dc, 0)


@functools.cache
def _sc_scatter_fn():
    return pl.kernel(
        _sc_body,
        out_type=[
            jax.ShapeDtypeStruct((NC, NP, D), jnp.float32),
            jax.ShapeDtypeStruct((NC, CB, D), jnp.float32),
        ],
        mesh=plsc.VectorSubcoreMesh(core_axis_name="c", subcore_axis_name="s",
                                    num_cores=NC, num_subcores=NS),
        compiler_params=pltpu.CompilerParams(needs_layout_passes=False),
        scratch_types=[
            pltpu.VMEM((K,), jnp.int32),
            pltpu.VMEM((K,), jnp.int32),
            pltpu.VMEM((L,), jnp.int32),
            pltpu.VMEM((K, D), jnp.float32),
            pltpu.VMEM((K, D), jnp.float32),
            pltpu.VMEM((K, D), jnp.float32),
            pltpu.VMEM((K, D), jnp.float32),
            pltpu.VMEM((CB, D), jnp.float32),
            pltpu.VMEM_SHARED((NP, D), jnp.float32),
            pltpu.SemaphoreType.DMA,
            pltpu.SemaphoreType.DMA,
            pltpu.SemaphoreType.DMA,
        ],
    )


def _update_body(x_ref, p_ref, c_ref, w2_ref, b2_ref, u1a_ref,
                 u1b_ref, bu1_ref, u2_ref, bu2_ref, out_ref):
    ssum = p_ref[0, :N_NODES] + p_ref[1, :N_NODES]
    cgrid = c_ref[0] + c_ref[1]
    vrow = lax.broadcasted_iota(jnp.int32, (N_NODES, CB), 0)
    hsel = (lax.shift_right_logical(vrow, 7)
            == lax.broadcasted_iota(jnp.int32, (N_NODES, CB), 1))
    rows = jnp.dot(hsel.astype(jnp.float32), cgrid,
                   preferred_element_type=jnp.float32, precision=_PREC)
    vlane = lax.broadcasted_iota(jnp.int32, (N_NODES, D), 0)
    lsel = (lax.bitwise_and(vlane, jnp.int32(127))
            == lax.broadcasted_iota(jnp.int32, (N_NODES, D), 1))
    cnt = jnp.sum(jnp.where(lsel, rows, 0.0), axis=1, keepdims=True)

    agg_sum = jnp.dot(ssum, w2_ref[...], preferred_element_type=jnp.float32,
                      precision=_PREC) + cnt * b2_ref[...]
    agg = agg_sum / jnp.maximum(cnt, 1.0)
    hu = jnp.dot(x_ref[...], u1a_ref[...], preferred_element_type=jnp.float32,
                 precision=_PREC)
    hu = hu + jnp.dot(agg, u1b_ref[...], preferred_element_type=jnp.float32,
                      precision=_PREC)
    hu = jnp.maximum(hu + bu1_ref[...], 0.0)
    out_ref[...] = jnp.dot(hu, u2_ref[...], preferred_element_type=jnp.float32,
                           precision=_PREC) + bu2_ref[...]


def _node_update(x, parts, cnts, w2, b2, u1a, u1b, bu1, u2, bu2):
    return pl.pallas_call(
        _update_body,
        out_shape=jax.ShapeDtypeStruct((N_NODES, D), jnp.float32),
    )(x, parts, cnts, w2, b2.reshape(1, D), u1a, u1b,
      bu1.reshape(1, D), u2, bu2.reshape(1, D))


def kernel(x, edge_index, edge_attr, W1, b1, W2, b2, U1, bu1, U2, bu2):
    row = edge_index[0].astype(jnp.int32)
    col = edge_index[1].astype(jnp.int32)
    npad = EP - N_EDGES
    row_p = jnp.concatenate([row, jnp.zeros((npad,), jnp.int32)])
    col_p = jnp.concatenate([col, jnp.full((npad,), NP - 1, jnp.int32)])
    x_pad = jnp.concatenate([x, jnp.zeros((NP - N_NODES, D), jnp.float32)])

    w1a, w1b, w1c = W1[:D], W1[D:2 * D], W1[2 * D:]
    u1a, u1b = U1[:D], U1[D:]

    xa, xb = _node_proj(x_pad, w1a, w1b)
    g = _edge_proj(edge_attr, w1c, b1)
    parts, cnts = _sc_scatter_fn()(row_p, col_p, xa, xb, g)
    return _node_update(x, parts, cnts, W2, b2, u1a, u1b, bu1, U2, bu2)


# speedup vs baseline: 2.3176x; 1.0083x over previous
import functools

import jax
import jax.numpy as jnp
from jax import lax
from jax.experimental import pallas as pl
from jax.experimental.pallas import tpu as pltpu
from jax.experimental.pallas import tpu_sc as plsc

N_NODES = 10000
N_EDGES = 320000
D = 128

NC = 2
NS = 16
L = 16
NW = NC * NS
NP = 10240
CB = NP // D
EP = 327680
EPW = EP // NW
K = 64
NCHUNK = EPW // K
IPL = 1024
CZ = NP // NS
ZCH = 64

_PREC = lax.Precision.HIGHEST


def _nodeproj_body(x_ref, wa_ref, wb_ref, xa_ref, xb_ref):
    xv = x_ref[...]
    xa_ref[...] = jnp.dot(xv, wa_ref[...], preferred_element_type=jnp.float32,
                          precision=_PREC)
    xb_ref[...] = jnp.dot(xv, wb_ref[...], preferred_element_type=jnp.float32,
                          precision=_PREC)


def _node_proj(x_pad, w1a, w1b):
    tn = 2048
    return pl.pallas_call(
        _nodeproj_body,
        grid=(NP // tn,),
        in_specs=[
            pl.BlockSpec((tn, D), lambda i: (i, 0)),
            pl.BlockSpec((D, D), lambda i: (0, 0)),
            pl.BlockSpec((D, D), lambda i: (0, 0)),
        ],
        out_specs=[
            pl.BlockSpec((tn, D), lambda i: (i, 0)),
            pl.BlockSpec((tn, D), lambda i: (i, 0)),
        ],
        out_shape=[
            jax.ShapeDtypeStruct((NP, D), jnp.float32),
            jax.ShapeDtypeStruct((NP, D), jnp.float32),
        ],
    )(x_pad, w1a, w1b)


def _edgeproj_body(e_ref, w_ref, b_ref, g_ref):
    g_ref[...] = jnp.dot(e_ref[...], w_ref[...], preferred_element_type=jnp.float32,
                         precision=_PREC) + b_ref[...]


def _edge_proj(edge_attr, w1c, b1):
    te = 2560
    n_in = N_EDGES // te
    return pl.pallas_call(
        _edgeproj_body,
        grid=(EP // te,),
        in_specs=[
            pl.BlockSpec((te, D), lambda i: (jnp.minimum(i, n_in - 1), 0)),
            pl.BlockSpec((D, D), lambda i: (0, 0)),
            pl.BlockSpec((1, D), lambda i: (0, 0)),
        ],
        out_specs=pl.BlockSpec((te, D), lambda i: (i, 0)),
        out_shape=jax.ShapeDtypeStruct((EP, D), jnp.float32),
    )(edge_attr, w1c, b1.reshape(1, D))


def _sc_body(row_hbm, col_hbm, xa_hbm, xb_hbm, g_hbm,
             out_hbm, cnt_hbm,
             rowa, cola, cidx,
             buf_a0, buf_b0, buf_g0, obuf0,
             cntv, acc, sem_g0, sem_g1, sem_s0):
    c = lax.axis_index("c")
    s = lax.axis_index("s")
    base = (c * NS + s) * EPW
    obuf = obuf0

    zv = jnp.zeros((L,), jnp.float32)

    def _zrow(i, _):
        for j in range(D // L):
            obuf[i, pl.ds(j * L, L)] = zv
        return 0
    lax.fori_loop(0, K, _zrow, 0)

    def _zacc(t, _):
        pltpu.sync_copy(obuf.at[pl.ds(0, ZCH)],
                        acc.at[pl.ds(s * CZ + t * ZCH, ZCH)])
        return 0
    lax.fori_loop(0, CZ // ZCH, _zacc, 0)

    def _zcnt(t, _):
        for j in range(D // L):
            cntv[t, pl.ds(j * L, L)] = zv
        return 0
    lax.fori_loop(0, CB, _zcnt, 0)

    plsc.subcore_barrier()

    ones = jnp.ones((L,), jnp.float32)

    def _chunk(ck, _):
        eo = base + ck * K
        pltpu.sync_copy(row_hbm.at[pl.ds(eo, K)], rowa)
        pltpu.sync_copy(col_hbm.at[pl.ds(eo, K)], cola)
        ca = pltpu.async_copy(xa_hbm.at[rowa], buf_a0, sem_g0)
        cb = pltpu.async_copy(xb_hbm.at[cola], buf_b0, sem_g1)
        cg = pltpu.async_copy(g_hbm.at[pl.ds(eo, K)], buf_g0, sem_s0)

        def _group(g, _):
            cv = cola[pl.ds(g * L, L)]
            plsc.addupdate_scatter(
                cntv, [lax.shift_right_logical(cv, 7),
                       lax.bitwise_and(cv, jnp.int32(127))], ones)
            return 0
        lax.fori_loop(0, K // L, _group, 0)

        ca.wait()
        cb.wait()
        cg.wait()

        def _crow(i, _):
            for j in range(D // L):
                sl = pl.ds(j * L, L)
                t = buf_a0[i, sl] + buf_b0[i, sl] + buf_g0[i, sl]
                obuf0[i, sl] = jnp.maximum(t, 0.0)
            return 0
        lax.fori_loop(0, K, _crow, 0)

        pltpu.sync_copy(obuf0, acc.at[cola], add=True)
        return 0
    lax.fori_loop(0, NCHUNK, _chunk, 0)

    plsc.subcore_barrier()

    def _dump(t, _):
        r0 = s * CZ + t * ZCH
        pltpu.sync_copy(acc.at[pl.ds(r0, ZCH)], obuf.at[pl.ds(0, ZCH)])
        pltpu.sync_copy(obuf.at[pl.ds(0, ZCH)], out_hbm.at[c, pl.ds(r0, ZCH)])
        return 0
    lax.fori_loop(0, CZ // ZCH, _dump, 0)

    plsc.subcore_barrier()

    @pl.when(s == 0)
    def _zero_cnt_rows():
        def _z2(i, _):
            for j in range(D // L):
                obuf[i, pl.ds(j * L, L)] = zv
            return 0
        lax.fori_loop(0, K, _z2, 0)

        def _zc(t, _):
            pltpu.sync_copy(obuf.at[pl.ds(0, CB // 2)],
                            acc.at[pl.ds(t * (CB // 2), CB // 2)])
            return 0
        lax.fori_loop(0, 2, _zc, 0)

    plsc.subcore_barrier()

    def _credu(h, _):
        cidx[...] = jnp.arange(L, dtype=jnp.int32) + h * L
        pltpu.sync_copy(cntv.at[pl.ds(h * L, L)], acc.at[cidx], add=True)
        return 0
    lax.fori_loop(0, CB // L, _credu, 0)
    plsc.subcore_barrier()

    @pl.when(s == 0)
    def _dump_cnt():
        def _dc(t, _):
            r0 = t * (CB // 2)
            pltpu.sync_copy(acc.at[pl.ds(r0, CB // 2)],
                            obuf.at[pl.ds(0, CB // 2)])
            pltpu.sync_copy(obuf.at[pl.ds(0, CB // 2)],
                            cnt_hbm.at[c, pl.ds(r0, CB // 2)])
            return 0
        lax.fori_loop(0, 2, _dc, 0)


@functools.cache
def _sc_scatter_fn():
    return pl.kernel(
        _sc_body,
        out_type=[
            jax.ShapeDtypeStruct((NC, NP, D), jnp.float32),
            jax.ShapeDtypeStruct((NC, CB, D), jnp.float32),
        ],
        mesh=plsc.VectorSubcoreMesh(core_axis_name="c", subcore_axis_name="s",
                                    num_cores=NC, num_subcores=NS),
        compiler_params=pltpu.CompilerParams(needs_layout_passes=False),
        scratch_types=[
            pltpu.VMEM((K,), jnp.int32),
            pltpu.VMEM((K,), jnp.int32),
            pltpu.VMEM((L,), jnp.int32),
            pltpu.VMEM((K, D), jnp.float32),
            pltpu.VMEM((K, D), jnp.float32),
            pltpu.VMEM((K, D), jnp.float32),
            pltpu.VMEM((K, D), jnp.float32),
            pltpu.VMEM((CB, D), jnp.float32),
            pltpu.VMEM_SHARED((NP, D), jnp.float32),
            pltpu.SemaphoreType.DMA,
            pltpu.SemaphoreType.DMA,
            pltpu.SemaphoreType.DMA,
        ],
    )


def _update_body(x_ref, p_ref, c_ref, w2_ref, b2_ref, u1a_ref,
                 u1b_ref, bu1_ref, u2_ref, bu2_ref, out_ref):
    ssum = p_ref[0, :N_NODES] + p_ref[1, :N_NODES]
    cgrid = c_ref[0] + c_ref[1]
    vrow = lax.broadcasted_iota(jnp.int32, (N_NODES, CB), 0)
    hsel = (lax.shift_right_logical(vrow, 7)
            == lax.broadcasted_iota(jnp.int32, (N_NODES, CB), 1))
    rows = jnp.dot(hsel.astype(jnp.float32), cgrid,
                   preferred_element_type=jnp.float32, precision=_PREC)
    vlane = lax.broadcasted_iota(jnp.int32, (N_NODES, D), 0)
    lsel = (lax.bitwise_and(vlane, jnp.int32(127))
            == lax.broadcasted_iota(jnp.int32, (N_NODES, D), 1))
    cnt = jnp.sum(jnp.where(lsel, rows, 0.0), axis=1, keepdims=True)

    agg_sum = jnp.dot(ssum, w2_ref[...], preferred_element_type=jnp.float32,
                      precision=_PREC) + cnt * b2_ref[...]
    agg = agg_sum / jnp.maximum(cnt, 1.0)
    hu = jnp.dot(x_ref[...], u1a_ref[...], preferred_element_type=jnp.float32,
                 precision=_PREC)
    hu = hu + jnp.dot(agg, u1b_ref[...], preferred_element_type=jnp.float32,
                      precision=_PREC)
    hu = jnp.maximum(hu + bu1_ref[...], 0.0)
    out_ref[...] = jnp.dot(hu, u2_ref[...], preferred_element_type=jnp.float32,
                           precision=_PREC) + bu2_ref[...]


def _node_update(x, parts, cnts, w2, b2, u1a, u1b, bu1, u2, bu2):
    return pl.pallas_call(
        _update_body,
        out_shape=jax.ShapeDtypeStruct((N_NODES, D), jnp.float32),
    )(x, parts, cnts, w2, b2.reshape(1, D), u1a, u1b,
      bu1.reshape(1, D), u2, bu2.reshape(1, D))


def kernel(x, edge_index, edge_attr, W1, b1, W2, b2, U1, bu1, U2, bu2):
    row = edge_index[0].astype(jnp.int32)
    col = edge_index[1].astype(jnp.int32)
    npad = EP - N_EDGES
    row_p = jnp.concatenate([row, jnp.zeros((npad,), jnp.int32)])
    col_p = jnp.concatenate([col, jnp.full((npad,), NP - 1, jnp.int32)])
    x_pad = jnp.concatenate([x, jnp.zeros((NP - N_NODES, D), jnp.float32)])

    w1a, w1b, w1c = W1[:D], W1[D:2 * D], W1[2 * D:]
    u1a, u1b = U1[:D], U1[D:]

    xa, xb = _node_proj(x_pad, w1a, w1b)
    g = _edge_proj(edge_attr, w1c, b1)
    parts, cnts = _sc_scatter_fn()(row_p, col_p, xa, xb, g)
    return _node_update(x, parts, cnts, W2, b2, u1a, u1b, bu1, U2, bu2)
